# baseline (speedup 1.0000x reference)
import functools

import jax
import jax.numpy as jnp
from jax import lax
from jax.experimental import pallas as pl
from jax.experimental.pallas import tpu as pltpu
from jax.experimental.pallas import tpu_sc as plsc

_NC, _NS, _L = 2, 16, 16
_NW = _NC * _NS
_D = 16
_G = 128
_GPC = 8
_CHUNK = _G * _GPC


def _gather_body(idx_hbm, table_hbm, out_hbm, idxv, rowsv, sem,
                 *, rows_per_worker):
    wid = lax.axis_index("s") * _NC + lax.axis_index("c")
    n_chunks = rows_per_worker // _CHUNK

    def chunk_body(c, _):
        r0 = wid * rows_per_worker + c * _CHUNK
        pltpu.sync_copy(idx_hbm.at[pl.ds(r0, _CHUNK)], idxv)
        for j in range(_GPC):
            pltpu.async_copy(table_hbm.at[idxv.at[pl.ds(j * _G, _G)]],
                             rowsv.at[pl.ds(j * _G, _G)], sem)
        for j in range(_GPC):
            pltpu.make_async_copy(table_hbm.at[idxv.at[pl.ds(j * _G, _G)]],
                                  rowsv.at[pl.ds(j * _G, _G)], sem).wait()
        pltpu.sync_copy(rowsv, out_hbm.at[pl.ds(r0, _CHUNK)])
        return 0

    lax.fori_loop(0, n_chunks, chunk_body, 0)


@jax.jit
def kernel(input, embedding_weight):
    B, H = input.shape
    n_emb, two_d = embedding_weight.shape
    d = two_d // 2
    n = B * H
    assert d == _D and n % (_NW * _CHUNK) == 0
    rows_per_worker = n // _NW
    table = embedding_weight.reshape(n_emb * 2, d)
    idx2 = (input.astype(jnp.int32) * 2).reshape(n)

    mesh = plsc.VectorSubcoreMesh(core_axis_name="c", subcore_axis_name="s")
    out = pl.kernel(
        functools.partial(_gather_body, rows_per_worker=rows_per_worker),
        out_type=jax.ShapeDtypeStruct((n, d), jnp.float32),
        mesh=mesh,
        compiler_params=pltpu.CompilerParams(use_tc_tiling_on_sc=False),
        scratch_types=[
            pltpu.VMEM((_CHUNK,), jnp.int32),
            pltpu.VMEM((_CHUNK, _D), jnp.float32),
            pltpu.SemaphoreType.DMA,
        ],
    )(idx2, table)
    return out.reshape(B, H, d)
